# baseline (speedup 1.0000x reference)
import functools

import jax
import jax.numpy as jnp
from jax import lax
from jax.experimental import pallas as pl
from jax.experimental.pallas import tpu as pltpu
from jax.experimental.pallas import tpu_sc as plsc

B = 1024
DB = 100000
NPAD = 102400
NCL = 1024
NSTEPS = NPAD // NCL
LG = NCL // 128
DPAD = 16


def _argmin_body(qd_ref, qn_ref, x_ref, in_ref, out_ref, acc_val, acc_idx):
    i = pl.program_id(0)

    @pl.when(i == 0)
    def _init():
        acc_val[...] = jnp.full((B, 128), jnp.inf, jnp.float32)
        acc_idx[...] = jnp.zeros((B, 128), jnp.int32)

    dots = lax.dot_general(
        qd_ref[...], x_ref[...],
        (((1,), (1,)), ((), ())),
        preferred_element_type=jnp.float32,
    )
    a = qn_ref[...] + in_ref[0]
    dist = a - dots

    lane = lax.broadcasted_iota(jnp.int32, (B, 128), 1)
    base = i * NCL
    av = acc_val[...]
    ai = acc_idx[...]
    for g in range(LG):
        d = dist[:, g * 128:(g + 1) * 128]
        idx = lane + (base + g * 128)
        m = d < av
        av = jnp.where(m, d, av)
        ai = jnp.where(m, idx, ai)
    acc_val[...] = av
    acc_idx[...] = ai

    @pl.when(i == NSTEPS - 1)
    def _final():
        row_min = jnp.min(av, axis=1, keepdims=True)
        cand = jnp.where(av == row_min, ai, jnp.int32(2147483647))
        out_ref[...] = jnp.min(cand, axis=1, keepdims=True)


def _nn_argmin(qd, qn, x_pad, in_pad):
    in3d = in_pad.reshape(NSTEPS, 1, NCL)
    return pl.pallas_call(
        _argmin_body,
        grid=(NSTEPS,),
        in_specs=[
            pl.BlockSpec((B, 3), lambda i: (0, 0)),
            pl.BlockSpec((B, 1), lambda i: (0, 0)),
            pl.BlockSpec((NCL, 3), lambda i: (i, 0)),
            pl.BlockSpec((1, 1, NCL), lambda i: (i, 0, 0)),
        ],
        out_specs=pl.BlockSpec((B, 1), lambda i: (0, 0)),
        out_shape=jax.ShapeDtypeStruct((B, 1), jnp.int32),
        scratch_shapes=[
            pltpu.VMEM((B, 128), jnp.float32),
            pltpu.VMEM((B, 128), jnp.int32),
        ],
    )(qd, qn, x_pad, in3d)


_SC_INFO = plsc.get_sparse_core_info()
_NW = _SC_INFO.num_cores * _SC_INFO.num_subcores
_B_PER_W = B // _NW


@functools.partial(
    pl.kernel,
    mesh=plsc.VectorSubcoreMesh(core_axis_name="c", subcore_axis_name="s"),
    out_type=jax.ShapeDtypeStruct((B, DPAD), jnp.float32),
    scratch_types=[
        pltpu.VMEM((_B_PER_W,), jnp.int32),
        pltpu.VMEM((_B_PER_W, DPAD), jnp.float32),
        pltpu.SemaphoreType.DMA,
    ],
    compiler_params=pltpu.CompilerParams(use_tc_tiling_on_sc=False),
)
def _sc_gather(table_hbm, idx_hbm, out_hbm, idx_v, rows_v, sem):
    wid = lax.axis_index("s") * _SC_INFO.num_cores + lax.axis_index("c")
    base = wid * _B_PER_W
    pltpu.sync_copy(idx_hbm.at[pl.ds(base, _B_PER_W)], idx_v)
    pltpu.async_copy(table_hbm.at[idx_v], rows_v, sem).wait()
    pltpu.sync_copy(rows_v, out_hbm.at[pl.ds(base, _B_PER_W)])


def kernel(query_vectors, temperatures, indices, s1, s2):
    dtype = jnp.float32
    orig_dtype = query_vectors.dtype
    q = lax.stop_gradient(query_vectors).astype(dtype)

    a = (s1 - s1.min()) / (s1.max() - s1.min()) + 1e-12
    b = (s2 - s2.min()) / (s2.max() - s2.min()) + 1e-12
    rm = jnp.concatenate([a, b], axis=-1)
    rm_pad = jnp.pad(rm, ((0, 0), (0, DPAD - 2)))

    qn = jnp.sum(q ** 2, axis=-1, keepdims=True)
    i_norm = jnp.sum(indices ** 2, axis=-1)

    qd = q + q
    x_pad = jnp.pad(indices.astype(dtype), ((0, NPAD - DB), (0, 0)))
    in_pad = jnp.pad(i_norm, (0, NPAD - DB),
                     constant_values=jnp.float32(jnp.inf))

    min_idx = _nn_argmin(qd, qn, x_pad, in_pad)
    gathered = _sc_gather(rm_pad, min_idx.reshape(B))
    return gathered[:, :2].astype(orig_dtype)


# speedup vs baseline: 2.1433x; 2.1433x over previous
import functools

import jax
import jax.numpy as jnp
from jax import lax
from jax.experimental import pallas as pl
from jax.experimental.pallas import tpu as pltpu
from jax.experimental.pallas import tpu_sc as plsc

B = 1024
DB = 100000
NPAD = 102400
NCL = 1024
NSTEPS = NPAD // NCL
LG = NCL // 128


def _argmin_body(qd_ref, qn_ref, xt_ref, in_ref, out_ref, acc_val, acc_idx):
    i = pl.program_id(0)

    @pl.when(i == 0)
    def _init():
        acc_val[...] = jnp.full((B, 128), jnp.inf, jnp.float32)
        acc_idx[...] = jnp.zeros((B, 128), jnp.int32)

    dots = lax.dot_general(
        qd_ref[...], xt_ref[...],
        (((1,), (0,)), ((), ())),
        preferred_element_type=jnp.float32,
    )
    lane = lax.broadcasted_iota(jnp.int32, (B, 128), 1)
    base = i * NCL
    qn = qn_ref[...]
    av = acc_val[...]
    ai = acc_idx[...]
    for g in range(LG):
        a = qn + in_ref[0, :, g * 128:(g + 1) * 128]
        d = a - dots[:, g * 128:(g + 1) * 128]
        idx = lane + (base + g * 128)
        m = d < av
        av = jnp.where(m, d, av)
        ai = jnp.where(m, idx, ai)
    acc_val[...] = av
    acc_idx[...] = ai

    @pl.when(i == NSTEPS - 1)
    def _final():
        row_min = jnp.min(av, axis=1, keepdims=True)
        cand = jnp.where(av == row_min, ai, jnp.int32(2147483647))
        out_ref[...] = jnp.min(cand, axis=1, keepdims=True)


def _nn_argmin(qd, qn, xt_pad, in_pad):
    in3d = in_pad.reshape(NSTEPS, 1, NCL)
    return pl.pallas_call(
        _argmin_body,
        grid=(NSTEPS,),
        in_specs=[
            pl.BlockSpec((B, 3), lambda i: (0, 0)),
            pl.BlockSpec((B, 1), lambda i: (0, 0)),
            pl.BlockSpec((3, NCL), lambda i: (0, i)),
            pl.BlockSpec((1, 1, NCL), lambda i: (i, 0, 0)),
        ],
        out_specs=pl.BlockSpec((B, 1), lambda i: (0, 0)),
        out_shape=jax.ShapeDtypeStruct((B, 1), jnp.int32),
        scratch_shapes=[
            pltpu.VMEM((B, 128), jnp.float32),
            pltpu.VMEM((B, 128), jnp.int32),
        ],
    )(qd, qn, xt_pad, in3d)


_SC_INFO = plsc.get_sparse_core_info()
_NW = _SC_INFO.num_cores * _SC_INFO.num_subcores
_B_PER_W = B // _NW


@functools.partial(
    pl.kernel,
    mesh=plsc.VectorSubcoreMesh(core_axis_name="c", subcore_axis_name="s"),
    out_type=jax.ShapeDtypeStruct((2, B), jnp.float32),
    scratch_types=[
        pltpu.VMEM((_B_PER_W,), jnp.int32),
        pltpu.VMEM((_B_PER_W,), jnp.float32),
        pltpu.VMEM((_B_PER_W,), jnp.float32),
        pltpu.SemaphoreType.DMA,
        pltpu.SemaphoreType.DMA,
    ],
    compiler_params=pltpu.CompilerParams(use_tc_tiling_on_sc=False),
)
def _sc_gather(s1_hbm, s2_hbm, idx_hbm, out_hbm, idx_v, g1_v, g2_v, sem1,
               sem2):
    wid = lax.axis_index("s") * _SC_INFO.num_cores + lax.axis_index("c")
    base = wid * _B_PER_W
    pltpu.sync_copy(idx_hbm.at[pl.ds(base, _B_PER_W)], idx_v)
    c1 = pltpu.async_copy(s1_hbm.at[idx_v], g1_v, sem1)
    c2 = pltpu.async_copy(s2_hbm.at[idx_v], g2_v, sem2)
    c1.wait()
    c2.wait()
    pltpu.sync_copy(g1_v, out_hbm.at[0, pl.ds(base, _B_PER_W)])
    pltpu.sync_copy(g2_v, out_hbm.at[1, pl.ds(base, _B_PER_W)])


def kernel(query_vectors, temperatures, indices, s1, s2):
    dtype = jnp.float32
    orig_dtype = query_vectors.dtype
    q = lax.stop_gradient(query_vectors).astype(dtype)

    qn = jnp.sum(q ** 2, axis=-1, keepdims=True)
    i_norm = jnp.sum(indices ** 2, axis=-1)

    qd = q + q
    xt = indices.T.astype(dtype)
    xt_pad = jnp.pad(xt, ((0, 0), (0, NPAD - DB)))
    in_pad = jnp.pad(i_norm, (0, NPAD - DB),
                     constant_values=jnp.float32(jnp.inf))

    min_idx = _nn_argmin(qd, qn, xt_pad, in_pad)

    g = _sc_gather(s1.reshape(DB), s2.reshape(DB), min_idx.reshape(B))
    a = (g[0] - s1.min()) / (s1.max() - s1.min()) + 1e-12
    b = (g[1] - s2.min()) / (s2.max() - s2.min()) + 1e-12
    out = jnp.stack([a, b], axis=-1)
    return out.astype(orig_dtype)


# speedup vs baseline: 2.2642x; 1.0564x over previous
import functools

import jax
import jax.numpy as jnp
from jax import lax
from jax.experimental import pallas as pl
from jax.experimental.pallas import tpu as pltpu
from jax.experimental.pallas import tpu_sc as plsc

B = 1024
DB = 100000
NPAD = 102400
NCL = 2048
NSTEPS = NPAD // NCL
LG = NCL // 128


def _argmin_body(qd_ref, qn_ref, xt_ref, in_ref, out_ref, acc_val, acc_idx):
    i = pl.program_id(0)

    @pl.when(i == 0)
    def _init():
        acc_val[...] = jnp.full((B, 128), jnp.inf, jnp.float32)
        acc_idx[...] = jnp.zeros((B, 128), jnp.int32)

    dots = lax.dot_general(
        qd_ref[...], xt_ref[...],
        (((1,), (0,)), ((), ())),
        preferred_element_type=jnp.float32,
    )
    lane = lax.broadcasted_iota(jnp.int32, (B, 128), 1)
    base = i * NCL
    qn = qn_ref[...]
    av = acc_val[...]
    ai = acc_idx[...]
    for g in range(LG):
        a = qn + in_ref[0, :, g * 128:(g + 1) * 128]
        d = a - dots[:, g * 128:(g + 1) * 128]
        idx = lane + (base + g * 128)
        m = d < av
        av = jnp.where(m, d, av)
        ai = jnp.where(m, idx, ai)
    acc_val[...] = av
    acc_idx[...] = ai

    @pl.when(i == NSTEPS - 1)
    def _final():
        row_min = jnp.min(av, axis=1, keepdims=True)
        cand = jnp.where(av == row_min, ai, jnp.int32(2147483647))
        out_ref[...] = jnp.min(cand, axis=1, keepdims=True)


def _nn_argmin(qd, qn, xt_pad, in_pad):
    in3d = in_pad.reshape(NSTEPS, 1, NCL)
    return pl.pallas_call(
        _argmin_body,
        grid=(NSTEPS,),
        in_specs=[
            pl.BlockSpec((B, 3), lambda i: (0, 0)),
            pl.BlockSpec((B, 1), lambda i: (0, 0)),
            pl.BlockSpec((3, NCL), lambda i: (0, i)),
            pl.BlockSpec((1, 1, NCL), lambda i: (i, 0, 0)),
        ],
        out_specs=pl.BlockSpec((B, 1), lambda i: (0, 0)),
        out_shape=jax.ShapeDtypeStruct((B, 1), jnp.int32),
        scratch_shapes=[
            pltpu.VMEM((B, 128), jnp.float32),
            pltpu.VMEM((B, 128), jnp.int32),
        ],
    )(qd, qn, xt_pad, in3d)


_SC_INFO = plsc.get_sparse_core_info()
_NW = _SC_INFO.num_cores * _SC_INFO.num_subcores
_B_PER_W = B // _NW


@functools.partial(
    pl.kernel,
    mesh=plsc.VectorSubcoreMesh(core_axis_name="c", subcore_axis_name="s"),
    out_type=jax.ShapeDtypeStruct((2, B), jnp.float32),
    scratch_types=[
        pltpu.VMEM((_B_PER_W,), jnp.int32),
        pltpu.VMEM((_B_PER_W,), jnp.float32),
        pltpu.VMEM((_B_PER_W,), jnp.float32),
        pltpu.SemaphoreType.DMA,
        pltpu.SemaphoreType.DMA,
    ],
    compiler_params=pltpu.CompilerParams(use_tc_tiling_on_sc=False),
)
def _sc_gather(s1_hbm, s2_hbm, idx_hbm, out_hbm, idx_v, g1_v, g2_v, sem1,
               sem2):
    wid = lax.axis_index("s") * _SC_INFO.num_cores + lax.axis_index("c")
    base = wid * _B_PER_W
    pltpu.sync_copy(idx_hbm.at[pl.ds(base, _B_PER_W)], idx_v)
    c1 = pltpu.async_copy(s1_hbm.at[idx_v], g1_v, sem1)
    c2 = pltpu.async_copy(s2_hbm.at[idx_v], g2_v, sem2)
    c1.wait()
    c2.wait()
    pltpu.sync_copy(g1_v, out_hbm.at[0, pl.ds(base, _B_PER_W)])
    pltpu.sync_copy(g2_v, out_hbm.at[1, pl.ds(base, _B_PER_W)])


def kernel(query_vectors, temperatures, indices, s1, s2):
    dtype = jnp.float32
    orig_dtype = query_vectors.dtype
    q = lax.stop_gradient(query_vectors).astype(dtype)

    qn = jnp.sum(q ** 2, axis=-1, keepdims=True)
    i_norm = jnp.sum(indices ** 2, axis=-1)

    qd = q + q
    xt = indices.T.astype(dtype)
    xt_pad = jnp.pad(xt, ((0, 0), (0, NPAD - DB)))
    in_pad = jnp.pad(i_norm, (0, NPAD - DB),
                     constant_values=jnp.float32(jnp.inf))

    min_idx = _nn_argmin(qd, qn, xt_pad, in_pad)

    g = _sc_gather(s1.reshape(DB), s2.reshape(DB), min_idx.reshape(B))
    a = (g[0] - s1.min()) / (s1.max() - s1.min()) + 1e-12
    b = (g[1] - s2.min()) / (s2.max() - s2.min()) + 1e-12
    out = jnp.stack([a, b], axis=-1)
    return out.astype(orig_dtype)


# speedup vs baseline: 2.3385x; 1.0328x over previous
import functools

import jax
import jax.numpy as jnp
from jax import lax
from jax.experimental import pallas as pl
from jax.experimental.pallas import tpu as pltpu
from jax.experimental.pallas import tpu_sc as plsc

B = 1024
DB = 100000
NPAD = 102400
NCL = 4096
NSTEPS = NPAD // NCL
LG = NCL // 128


def _argmin_body(qd_ref, qn_ref, xt_ref, in_ref, out_ref, acc_val, acc_idx):
    i = pl.program_id(0)

    @pl.when(i == 0)
    def _init():
        acc_val[...] = jnp.full((B, 128), jnp.inf, jnp.float32)
        acc_idx[...] = jnp.zeros((B, 128), jnp.int32)

    dots = lax.dot_general(
        qd_ref[...], xt_ref[...],
        (((1,), (0,)), ((), ())),
        preferred_element_type=jnp.float32,
    )
    lane = lax.broadcasted_iota(jnp.int32, (B, 128), 1)
    base = i * NCL
    qn = qn_ref[...]
    av = acc_val[...]
    ai = acc_idx[...]
    for g in range(LG):
        a = qn + in_ref[0, :, g * 128:(g + 1) * 128]
        d = a - dots[:, g * 128:(g + 1) * 128]
        idx = lane + (base + g * 128)
        m = d < av
        av = jnp.where(m, d, av)
        ai = jnp.where(m, idx, ai)
    acc_val[...] = av
    acc_idx[...] = ai

    @pl.when(i == NSTEPS - 1)
    def _final():
        row_min = jnp.min(av, axis=1, keepdims=True)
        cand = jnp.where(av == row_min, ai, jnp.int32(2147483647))
        out_ref[...] = jnp.min(cand, axis=1, keepdims=True)


def _nn_argmin(qd, qn, xt_pad, in_pad):
    in3d = in_pad.reshape(NSTEPS, 1, NCL)
    return pl.pallas_call(
        _argmin_body,
        grid=(NSTEPS,),
        in_specs=[
            pl.BlockSpec((B, 3), lambda i: (0, 0)),
            pl.BlockSpec((B, 1), lambda i: (0, 0)),
            pl.BlockSpec((3, NCL), lambda i: (0, i)),
            pl.BlockSpec((1, 1, NCL), lambda i: (i, 0, 0)),
        ],
        out_specs=pl.BlockSpec((B, 1), lambda i: (0, 0)),
        out_shape=jax.ShapeDtypeStruct((B, 1), jnp.int32),
        scratch_shapes=[
            pltpu.VMEM((B, 128), jnp.float32),
            pltpu.VMEM((B, 128), jnp.int32),
        ],
    )(qd, qn, xt_pad, in3d)


_SC_INFO = plsc.get_sparse_core_info()
_NW = _SC_INFO.num_cores * _SC_INFO.num_subcores
_B_PER_W = B // _NW


@functools.partial(
    pl.kernel,
    mesh=plsc.VectorSubcoreMesh(core_axis_name="c", subcore_axis_name="s"),
    out_type=jax.ShapeDtypeStruct((2, B), jnp.float32),
    scratch_types=[
        pltpu.VMEM((_B_PER_W,), jnp.int32),
        pltpu.VMEM((_B_PER_W,), jnp.float32),
        pltpu.VMEM((_B_PER_W,), jnp.float32),
        pltpu.SemaphoreType.DMA,
        pltpu.SemaphoreType.DMA,
    ],
    compiler_params=pltpu.CompilerParams(use_tc_tiling_on_sc=False),
)
def _sc_gather(s1_hbm, s2_hbm, idx_hbm, out_hbm, idx_v, g1_v, g2_v, sem1,
               sem2):
    wid = lax.axis_index("s") * _SC_INFO.num_cores + lax.axis_index("c")
    base = wid * _B_PER_W
    pltpu.sync_copy(idx_hbm.at[pl.ds(base, _B_PER_W)], idx_v)
    c1 = pltpu.async_copy(s1_hbm.at[idx_v], g1_v, sem1)
    c2 = pltpu.async_copy(s2_hbm.at[idx_v], g2_v, sem2)
    c1.wait()
    c2.wait()
    pltpu.sync_copy(g1_v, out_hbm.at[0, pl.ds(base, _B_PER_W)])
    pltpu.sync_copy(g2_v, out_hbm.at[1, pl.ds(base, _B_PER_W)])


def kernel(query_vectors, temperatures, indices, s1, s2):
    dtype = jnp.float32
    orig_dtype = query_vectors.dtype
    q = lax.stop_gradient(query_vectors).astype(dtype)

    qn = jnp.sum(q ** 2, axis=-1, keepdims=True)
    i_norm = jnp.sum(indices ** 2, axis=-1)

    qd = q + q
    xt = indices.T.astype(dtype)
    xt_pad = jnp.pad(xt, ((0, 0), (0, NPAD - DB)))
    in_pad = jnp.pad(i_norm, (0, NPAD - DB),
                     constant_values=jnp.float32(jnp.inf))

    min_idx = _nn_argmin(qd, qn, xt_pad, in_pad)

    g = _sc_gather(s1.reshape(DB), s2.reshape(DB), min_idx.reshape(B))
    a = (g[0] - s1.min()) / (s1.max() - s1.min()) + 1e-12
    b = (g[1] - s2.min()) / (s2.max() - s2.min()) + 1e-12
    out = jnp.stack([a, b], axis=-1)
    return out.astype(orig_dtype)


# speedup vs baseline: 2.3488x; 1.0044x over previous
import functools

import jax
import jax.numpy as jnp
from jax import lax
from jax.experimental import pallas as pl
from jax.experimental.pallas import tpu as pltpu
from jax.experimental.pallas import tpu_sc as plsc

B = 1024
DB = 100000
NPAD = 102400
NCL = 4096
NSTEPS = NPAD // NCL
LG = NCL // 128
QB = 128
NQB = B // QB


def _argmin_body(qd_ref, qn_ref, xt_ref, in_ref, out_ref, acc_val, acc_idx):
    i = pl.program_id(0)

    @pl.when(i == 0)
    def _init():
        acc_val[...] = jnp.full((B, 128), jnp.inf, jnp.float32)
        acc_idx[...] = jnp.zeros((B, 128), jnp.int32)

    dots = lax.dot_general(
        qd_ref[...], xt_ref[...],
        (((1,), (0,)), ((), ())),
        preferred_element_type=jnp.float32,
    )
    lane = lax.broadcasted_iota(jnp.int32, (QB, 128), 1)
    base = i * NCL
    for qb in range(NQB):
        rs = slice(qb * QB, (qb + 1) * QB)
        qn = qn_ref[rs, :]
        av = acc_val[rs, :]
        ai = acc_idx[rs, :]
        for g in range(LG):
            a = qn + in_ref[0, :, g * 128:(g + 1) * 128]
            d = a - dots[rs, g * 128:(g + 1) * 128]
            idx = lane + (base + g * 128)
            m = d < av
            av = jnp.where(m, d, av)
            ai = jnp.where(m, idx, ai)
        acc_val[rs, :] = av
        acc_idx[rs, :] = ai

    @pl.when(i == NSTEPS - 1)
    def _final():
        fav = acc_val[...]
        fai = acc_idx[...]
        row_min = jnp.min(fav, axis=1, keepdims=True)
        cand = jnp.where(fav == row_min, fai, jnp.int32(2147483647))
        out_ref[...] = jnp.min(cand, axis=1, keepdims=True)


def _nn_argmin(qd, qn, xt_pad, in_pad):
    in3d = in_pad.reshape(NSTEPS, 1, NCL)
    return pl.pallas_call(
        _argmin_body,
        grid=(NSTEPS,),
        in_specs=[
            pl.BlockSpec((B, 3), lambda i: (0, 0)),
            pl.BlockSpec((B, 1), lambda i: (0, 0)),
            pl.BlockSpec((3, NCL), lambda i: (0, i)),
            pl.BlockSpec((1, 1, NCL), lambda i: (i, 0, 0)),
        ],
        out_specs=pl.BlockSpec((B, 1), lambda i: (0, 0)),
        out_shape=jax.ShapeDtypeStruct((B, 1), jnp.int32),
        scratch_shapes=[
            pltpu.VMEM((B, 128), jnp.float32),
            pltpu.VMEM((B, 128), jnp.int32),
        ],
    )(qd, qn, xt_pad, in3d)


_SC_INFO = plsc.get_sparse_core_info()
_NW = _SC_INFO.num_cores * _SC_INFO.num_subcores
_B_PER_W = B // _NW


@functools.partial(
    pl.kernel,
    mesh=plsc.VectorSubcoreMesh(core_axis_name="c", subcore_axis_name="s"),
    out_type=jax.ShapeDtypeStruct((2, B), jnp.float32),
    scratch_types=[
        pltpu.VMEM((_B_PER_W,), jnp.int32),
        pltpu.VMEM((_B_PER_W,), jnp.float32),
        pltpu.VMEM((_B_PER_W,), jnp.float32),
        pltpu.SemaphoreType.DMA,
        pltpu.SemaphoreType.DMA,
    ],
    compiler_params=pltpu.CompilerParams(use_tc_tiling_on_sc=False),
)
def _sc_gather(s1_hbm, s2_hbm, idx_hbm, out_hbm, idx_v, g1_v, g2_v, sem1,
               sem2):
    wid = lax.axis_index("s") * _SC_INFO.num_cores + lax.axis_index("c")
    base = wid * _B_PER_W
    pltpu.sync_copy(idx_hbm.at[pl.ds(base, _B_PER_W)], idx_v)
    c1 = pltpu.async_copy(s1_hbm.at[idx_v], g1_v, sem1)
    c2 = pltpu.async_copy(s2_hbm.at[idx_v], g2_v, sem2)
    c1.wait()
    c2.wait()
    pltpu.sync_copy(g1_v, out_hbm.at[0, pl.ds(base, _B_PER_W)])
    pltpu.sync_copy(g2_v, out_hbm.at[1, pl.ds(base, _B_PER_W)])


def kernel(query_vectors, temperatures, indices, s1, s2):
    dtype = jnp.float32
    orig_dtype = query_vectors.dtype
    q = lax.stop_gradient(query_vectors).astype(dtype)

    qn = jnp.sum(q ** 2, axis=-1, keepdims=True)
    i_norm = jnp.sum(indices ** 2, axis=-1)

    qd = q + q
    xt = indices.T.astype(dtype)
    xt_pad = jnp.pad(xt, ((0, 0), (0, NPAD - DB)))
    in_pad = jnp.pad(i_norm, (0, NPAD - DB),
                     constant_values=jnp.float32(jnp.inf))

    min_idx = _nn_argmin(qd, qn, xt_pad, in_pad)

    g = _sc_gather(s1.reshape(DB), s2.reshape(DB), min_idx.reshape(B))
    a = (g[0] - s1.min()) / (s1.max() - s1.min()) + 1e-12
    b = (g[1] - s2.min()) / (s2.max() - s2.min()) + 1e-12
    out = jnp.stack([a, b], axis=-1)
    return out.astype(orig_dtype)
